# baseline (speedup 1.0000x reference)
import functools

import jax
import jax.numpy as jnp
from jax import lax
from jax.experimental import pallas as pl
from jax.experimental.pallas import tpu as pltpu
from jax.experimental.pallas import tpu_sc as plsc

DIM = 128
NSEG = 65
NW = 32
CH = 112
LANES = 16
NCHK = DIM // LANES


def _h_matmul_kernel(x_ref, w_ref, b_ref, o_ref):
    o_ref[...] = lax.dot_general(
        x_ref[...], w_ref[...], (((1,), (1,)), ((), ())),
        preferred_element_type=jnp.float32) + b_ref[...]


def _combine_kernel(ps_ref, pm_ref, attn_ref, o_ref):
    s = jnp.sum(ps_ref[...], axis=0)
    m = jnp.max(pm_ref[...], axis=0)
    attn = attn_ref[...]
    tstar = m * jnp.where(attn >= 0.0, 1.0, -1.0)
    z = jnp.where(tstar > 0.0, tstar, jnp.exp(tstar) - 1.0)
    za = z * attn
    r = jnp.where(za > 0.0, za, jnp.exp(za) - 1.0)
    o_ref[...] = jnp.concatenate([s[:NSEG - 1], r[:NSEG - 1]], axis=1)


def _sc_edge_kernel(nchunk, h_hbm, src_hbm, dst_hbm, seg_hbm, attn_hbm,
                    wr_hbm, br_hbm, fp_hbm, batch_hbm, outsum_hbm, outmax_hbm,
                    srcv0, dstv0, segv0, srcv1, dstv1, segv1,
                    srcv2, dstv2, segv2, srcv3, dstv3, segv3,
                    rs0, rd0, rs1, rd1, accs, accm0, accm1, accm2, accm3,
                    rbuf, attnv, wrv, brv, fpv, batchv,
                    semS0, semD0, semS1, semD1,
                    semI0, semI1, semI2, semI3):
    cid = lax.axis_index("c")
    sid = lax.axis_index("s")
    wid = sid * 2 + cid
    epw = nchunk * CH
    base = wid * epw

    pltpu.sync_copy(attn_hbm, attnv)
    pltpu.sync_copy(wr_hbm, wrv)
    pltpu.sync_copy(br_hbm, brv)
    pltpu.sync_copy(fp_hbm, fpv)
    pltpu.sync_copy(batch_hbm, batchv)

    zero16 = jnp.zeros((LANES,), jnp.float32)
    ninf16 = jnp.full((LANES,), -jnp.inf, jnp.float32)

    accms = (accm0, accm1, accm2, accm3)

    def init_body(i, carry):
        for j in range(NCHK):
            sl = pl.ds(LANES * j, LANES)
            accs[i, sl] = zero16
            for am in accms:
                am[i, sl] = ninf16
        return carry

    lax.fori_loop(0, NSEG, init_body, 0)
    brsc = brv[...][0]
    attn_r = [attnv[pl.ds(LANES * j, LANES)] for j in range(NCHK)]
    wr_r = [wrv[pl.ds(LANES * j, LANES)] for j in range(NCHK)]
    sgn_r = [jnp.where(a >= 0.0, 1.0, -1.0).astype(jnp.float32)
             for a in attn_r]

    idx_bufs = ((srcv0, dstv0, segv0), (srcv1, dstv1, segv1),
                (srcv2, dstv2, segv2), (srcv3, dstv3, segv3))
    dat_bufs = ((rs0, rd0), (rs1, rd1))
    dat_sems = ((semS0, semD0), (semS1, semD1))
    idx_sems = (semI0, semI1, semI2, semI3)

    def load_idx(t, iset):
        off = base + t * CH
        sv, dv, gv = idx_bufs[iset]
        pltpu.async_copy(src_hbm.at[pl.ds(off, CH)], sv, idx_sems[iset])
        pltpu.async_copy(dst_hbm.at[pl.ds(off, CH)], dv, idx_sems[iset])
        pltpu.async_copy(seg_hbm.at[pl.ds(off, CH)], gv, idx_sems[iset])

    def fire_gather(dset, iset):
        sv, dv, _ = idx_bufs[iset]
        rs, rd = dat_bufs[dset]
        sA, sB = dat_sems[dset]
        pltpu.async_copy(h_hbm.at[sv], rs, sA)
        pltpu.async_copy(h_hbm.at[dv], rd, sB)

    def wait_gather(dset, iset):
        sv, dv, _ = idx_bufs[iset]
        rs, rd = dat_bufs[dset]
        sA, sB = dat_sems[dset]
        pltpu.make_async_copy(h_hbm.at[sv], rs, sA).wait()
        pltpu.make_async_copy(h_hbm.at[dv], rd, sB).wait()

    def wait_idx_set(iset):
        sv, dv, gv = idx_bufs[iset]
        off = pl.ds(0, CH)
        pltpu.make_async_copy(src_hbm.at[off], sv, idx_sems[iset]).wait()
        pltpu.make_async_copy(dst_hbm.at[off], dv, idx_sems[iset]).wait()
        pltpu.make_async_copy(seg_hbm.at[off], gv, idx_sems[iset]).wait()

    def compute_chunk(dset, iset):
        rs, rd = dat_bufs[dset]
        sv_b, dv_b, gv = idx_bufs[iset]

        def group_body(g, gcarry):
            gsl = pl.ds(g * LANES, LANES)
            lovec = sv_b[gsl]
            hivec = dv_b[gsl]
            vvec = gv[gsl]
            fpl = plsc.load_gather(fpv, [lovec])
            fph = plsc.load_gather(fpv, [hivec])
            uvec = jnp.where(fpl <= fph, lovec, hivec)
            segb = plsc.load_gather(batchv, [uvec])
            segvec = jnp.where(vvec != 0, segb, NSEG - 1)
            def load_t(p):
                e0 = g * LANES + 2 * p
                e1 = e0 + 1
                t0 = []
                t1 = []
                for j in range(NCHK):
                    sl = pl.ds(LANES * j, LANES)
                    t0.append(rs[e0, sl] + rd[e0, sl])
                    t1.append(rs[e1, sl] + rd[e1, sl])
                return t0, t1

            cur = load_t(0)
            for p in range(LANES // 2):
                t0, t1 = cur
                seg0 = segvec[2 * p]
                seg1 = segvec[2 * p + 1]
                m0 = [accm0[seg0, pl.ds(LANES * j, LANES)]
                      for j in range(NCHK)]
                m1 = [accm1[seg1, pl.ds(LANES * j, LANES)]
                      for j in range(NCHK)]
                for j in range(NCHK):
                    sl = pl.ds(LANES * j, LANES)
                    accm0[seg0, sl] = jnp.maximum(m0[j], t0[j] * sgn_r[j])
                    accm1[seg1, sl] = jnp.maximum(m1[j], t1[j] * sgn_r[j])
                if p + 1 < LANES // 2:
                    cur = load_t(p + 1)
                r0 = []
                r1 = []
                dot0 = zero16
                dot1 = zero16
                for j in range(NCHK):
                    z0 = jnp.where(t0[j] > 0.0, t0[j],
                                   jnp.exp(t0[j]) - 1.0)
                    z1 = jnp.where(t1[j] > 0.0, t1[j],
                                   jnp.exp(t1[j]) - 1.0)
                    za0 = z0 * attn_r[j]
                    za1 = z1 * attn_r[j]
                    ra = jnp.where(za0 > 0.0, za0, jnp.exp(za0) - 1.0)
                    rb = jnp.where(za1 > 0.0, za1, jnp.exp(za1) - 1.0)
                    r0.append(ra)
                    r1.append(rb)
                    dot0 = dot0 + ra * wr_r[j]
                    dot1 = dot1 + rb * wr_r[j]
                d0 = jnp.sum(dot0) + brsc
                d1 = jnp.sum(dot1) + brsc
                s0 = 1.0 / (1.0 + jnp.exp(
                    jnp.full((LANES,), -d0, jnp.float32)))
                s1 = 1.0 / (1.0 + jnp.exp(
                    jnp.full((LANES,), -d1, jnp.float32)))
                for j in range(NCHK):
                    sl = pl.ds(LANES * j, LANES)
                    plsc.addupdate(accs.at[seg0, sl], s0 * r0[j])
                    plsc.addupdate(accs.at[seg1, sl], s1 * r1[j])
            return gcarry

        lax.fori_loop(0, CH // LANES, group_body, 0)

    load_idx(0, 0)
    load_idx(1, 1)
    wait_idx_set(0)
    fire_gather(0, 0)

    def pair_body(p, carry):
        for b in (0, 1):
            t = 2 * p + b
            nxt = 1 - b
            wait_idx_set(nxt)
            fire_gather(nxt, nxt)
            wait_gather(b, b)
            compute_chunk(b, b)
            load_idx(t + 2, b)
        return carry

    lax.fori_loop(0, nchunk // 2, pair_body, 0)
    wait_idx_set(1)
    wait_gather(0, 0)

    def merge_body(i, carry):
        for j in range(NCHK):
            sl = pl.ds(LANES * j, LANES)
            accm0[i, sl] = jnp.maximum(
                jnp.maximum(accm0[i, sl], accm1[i, sl]),
                jnp.maximum(accm2[i, sl], accm3[i, sl]))
        return carry

    lax.fori_loop(0, NSEG, merge_body, 0)
    pltpu.sync_copy(accs, outsum_hbm.at[wid])
    pltpu.sync_copy(accm0, outmax_hbm.at[wid])


def kernel(x, edge_index, edge_attr, pos, batch, W, b, attn, Wr, br):
    del edge_attr, pos
    E = edge_index.shape[1]
    N = x.shape[0]

    a = edge_index[0].astype(jnp.int32)
    b_ = edge_index[1].astype(jnp.int32)
    idx = jnp.arange(E, dtype=jnp.int32)
    first_pos = (jnp.full((N,), 2 * E, jnp.int32)
                 .at[a].min(2 * idx).at[b_].min(2 * idx + 1))
    lo = jnp.minimum(a, b_)
    hi = jnp.maximum(a, b_)
    key_s = jnp.sort(lo * N + hi)
    keep = jnp.concatenate([jnp.ones((1,), bool), key_s[1:] != key_s[:-1]])
    lo_s = key_s // N
    hi_s = key_s - lo_s * N
    batch32 = batch.astype(jnp.int32)
    node_ids = jnp.arange(N, dtype=jnp.int32)
    iso = first_pos == 2 * E
    total = E + N
    epw = -(-total // (NW * 2 * CH)) * 2 * CH
    npad = NW * epw - total + 2 * CH
    pad_idx = (jnp.arange(npad, dtype=jnp.int32) * 97) % N
    src_all = jnp.concatenate([lo_s, node_ids, pad_idx])
    dst_all = jnp.concatenate([hi_s, node_ids, pad_idx])
    valid_all = jnp.concatenate(
        [keep.astype(jnp.int32), iso.astype(jnp.int32),
         jnp.zeros((npad,), jnp.int32)])

    h = pl.pallas_call(
        _h_matmul_kernel,
        out_shape=jax.ShapeDtypeStruct((N, DIM), jnp.float32),
    )(x, W, b.reshape(1, DIM))

    nchunk = epw // CH
    mesh = plsc.VectorSubcoreMesh(
        core_axis_name="c", subcore_axis_name="s", num_cores=2,
        num_subcores=16)
    sc_fn = functools.partial(
        pl.kernel,
        out_type=[jax.ShapeDtypeStruct((NW, NSEG, DIM), jnp.float32),
                  jax.ShapeDtypeStruct((NW, NSEG, DIM), jnp.float32)],
        mesh=mesh,
        scratch_types=[
            pltpu.VMEM((CH,), jnp.int32),
            pltpu.VMEM((CH,), jnp.int32),
            pltpu.VMEM((CH,), jnp.int32),
            pltpu.VMEM((CH,), jnp.int32),
            pltpu.VMEM((CH,), jnp.int32),
            pltpu.VMEM((CH,), jnp.int32),
            pltpu.VMEM((CH,), jnp.int32),
            pltpu.VMEM((CH,), jnp.int32),
            pltpu.VMEM((CH,), jnp.int32),
            pltpu.VMEM((CH,), jnp.int32),
            pltpu.VMEM((CH,), jnp.int32),
            pltpu.VMEM((CH,), jnp.int32),
            pltpu.VMEM((CH, DIM), jnp.float32),
            pltpu.VMEM((CH, DIM), jnp.float32),
            pltpu.VMEM((CH, DIM), jnp.float32),
            pltpu.VMEM((CH, DIM), jnp.float32),
            pltpu.VMEM((NSEG, DIM), jnp.float32),
            pltpu.VMEM((NSEG, DIM), jnp.float32),
            pltpu.VMEM((NSEG, DIM), jnp.float32),
            pltpu.VMEM((NSEG, DIM), jnp.float32),
            pltpu.VMEM((NSEG, DIM), jnp.float32),
            pltpu.VMEM((4, DIM), jnp.float32),
            pltpu.VMEM((DIM,), jnp.float32),
            pltpu.VMEM((DIM,), jnp.float32),
            pltpu.VMEM((LANES,), jnp.float32),
            pltpu.VMEM((N,), jnp.int32),
            pltpu.VMEM((N,), jnp.int32),
            pltpu.SemaphoreType.DMA,
            pltpu.SemaphoreType.DMA,
            pltpu.SemaphoreType.DMA,
            pltpu.SemaphoreType.DMA,
            pltpu.SemaphoreType.DMA,
            pltpu.SemaphoreType.DMA,
            pltpu.SemaphoreType.DMA,
            pltpu.SemaphoreType.DMA,
        ],
        compiler_params=pltpu.CompilerParams(needs_layout_passes=False),
    )(functools.partial(_sc_edge_kernel, nchunk))
    attn_r = attn.reshape(DIM).astype(jnp.float32)
    wr_r = Wr.reshape(DIM).astype(jnp.float32)
    br_p = jnp.broadcast_to(br.astype(jnp.float32), (LANES,))
    psum, pmax = sc_fn(h, src_all, dst_all, valid_all, attn_r, wr_r, br_p,
                       first_pos, batch32)

    out = pl.pallas_call(
        _combine_kernel,
        out_shape=jax.ShapeDtypeStruct((NSEG - 1, 2 * DIM), jnp.float32),
    )(psum, pmax, attn.reshape(1, DIM).astype(jnp.float32))
    return out


# speedup vs baseline: 1.0774x; 1.0774x over previous
import functools

import jax
import jax.numpy as jnp
from jax import lax
from jax.experimental import pallas as pl
from jax.experimental.pallas import tpu as pltpu
from jax.experimental.pallas import tpu_sc as plsc

DIM = 128
NSEG = 65
NW = 32
CH = 128
LANES = 16
NCHK = DIM // LANES


def _h_matmul_kernel(x_ref, w_ref, b_ref, o_ref):
    o_ref[...] = lax.dot_general(
        x_ref[...], w_ref[...], (((1,), (1,)), ((), ())),
        preferred_element_type=jnp.float32) + b_ref[...]


def _combine_kernel(ps_ref, pm_ref, attn_ref, o_ref):
    s = jnp.sum(ps_ref[...], axis=0)
    m = jnp.max(pm_ref[...], axis=0)
    attn = attn_ref[...]
    tstar = m * jnp.where(attn >= 0.0, 1.0, -1.0)
    z = jnp.where(tstar > 0.0, tstar, jnp.exp(tstar) - 1.0)
    za = z * attn
    r = jnp.where(za > 0.0, za, jnp.exp(za) - 1.0)
    o_ref[...] = jnp.concatenate([s[:NSEG - 1], r[:NSEG - 1]], axis=1)


def _sc_edge_kernel(nchunk, h_hbm, src_hbm, dst_hbm, seg_hbm, attn_hbm,
                    wr_hbm, br_hbm, fp_hbm, batch_hbm, outsum_hbm, outmax_hbm,
                    srcv0, dstv0, segv0, srcv1, dstv1, segv1,
                    rs0, rd0, rs1, rd1, accs, accm0, accm1,
                    attnv, wrv, brv, fpv, batchv,
                    semS0, semD0, semS1, semD1, semI0, semI1):
    cid = lax.axis_index("c")
    sid = lax.axis_index("s")
    wid = sid * 2 + cid
    epw = nchunk * CH
    base = wid * epw

    pltpu.sync_copy(attn_hbm, attnv)
    pltpu.sync_copy(wr_hbm, wrv)
    pltpu.sync_copy(br_hbm, brv)
    pltpu.sync_copy(fp_hbm, fpv)
    pltpu.sync_copy(batch_hbm, batchv)

    zero16 = jnp.zeros((LANES,), jnp.float32)
    ninf16 = jnp.full((LANES,), -jnp.inf, jnp.float32)

    def init_body(i, carry):
        for j in range(NCHK):
            sl = pl.ds(LANES * j, LANES)
            accs[i, sl] = zero16
            accm0[i, sl] = ninf16
            accm1[i, sl] = ninf16
        return carry

    lax.fori_loop(0, NSEG, init_body, 0)
    brsc = brv[...][0]
    attn_r = [attnv[pl.ds(LANES * j, LANES)] for j in range(NCHK)]
    wr_r = [wrv[pl.ds(LANES * j, LANES)] for j in range(NCHK)]
    sgn_r = [jnp.where(a >= 0.0, 1.0, -1.0).astype(jnp.float32)
             for a in attn_r]

    idx_bufs = ((srcv0, dstv0, segv0), (srcv1, dstv1, segv1))
    dat_bufs = ((rs0, rd0), (rs1, rd1))
    dat_sems = ((semS0, semD0), (semS1, semD1))
    idx_sems = (semI0, semI1)

    def load_idx(t, iset):
        off = base + t * CH
        sv, dv, gv = idx_bufs[iset]
        pltpu.async_copy(src_hbm.at[pl.ds(off, CH)], sv, idx_sems[iset])
        pltpu.async_copy(dst_hbm.at[pl.ds(off, CH)], dv, idx_sems[iset])
        pltpu.async_copy(seg_hbm.at[pl.ds(off, CH)], gv, idx_sems[iset])

    def fire_gather(dset, iset):
        sv, dv, _ = idx_bufs[iset]
        rs, rd = dat_bufs[dset]
        sA, sB = dat_sems[dset]
        pltpu.async_copy(h_hbm.at[sv], rs, sA)
        pltpu.async_copy(h_hbm.at[dv], rd, sB)

    def wait_gather(dset, iset):
        sv, dv, _ = idx_bufs[iset]
        rs, rd = dat_bufs[dset]
        sA, sB = dat_sems[dset]
        pltpu.make_async_copy(h_hbm.at[sv], rs, sA).wait()
        pltpu.make_async_copy(h_hbm.at[dv], rd, sB).wait()

    def wait_idx_set(iset):
        sv, dv, gv = idx_bufs[iset]
        off = pl.ds(0, CH)
        pltpu.make_async_copy(src_hbm.at[off], sv, idx_sems[iset]).wait()
        pltpu.make_async_copy(dst_hbm.at[off], dv, idx_sems[iset]).wait()
        pltpu.make_async_copy(seg_hbm.at[off], gv, idx_sems[iset]).wait()

    def compute_chunk(dset, iset):
        rs, rd = dat_bufs[dset]
        sv_b, dv_b, gv = idx_bufs[iset]

        def group_body(g, gcarry):
            gsl = pl.ds(g * LANES, LANES)
            lovec = sv_b[gsl]
            hivec = dv_b[gsl]
            vvec = gv[gsl]
            fpl = plsc.load_gather(fpv, [lovec])
            fph = plsc.load_gather(fpv, [hivec])
            uvec = jnp.where(fpl <= fph, lovec, hivec)
            segb = plsc.load_gather(batchv, [uvec])
            segvec = jnp.where(vvec != 0, segb, NSEG - 1)
            for p in range(LANES // 2):
                e0 = g * LANES + 2 * p
                e1 = e0 + 1
                seg0 = segvec[2 * p]
                seg1 = segvec[2 * p + 1]
                t0 = []
                t1 = []
                for j in range(NCHK):
                    sl = pl.ds(LANES * j, LANES)
                    t0.append(rs[e0, sl] + rd[e0, sl])
                    t1.append(rs[e1, sl] + rd[e1, sl])
                m0 = [accm0[seg0, pl.ds(LANES * j, LANES)]
                      for j in range(NCHK)]
                m1 = [accm1[seg1, pl.ds(LANES * j, LANES)]
                      for j in range(NCHK)]
                for j in range(NCHK):
                    sl = pl.ds(LANES * j, LANES)
                    accm0[seg0, sl] = jnp.maximum(m0[j], t0[j] * sgn_r[j])
                    accm1[seg1, sl] = jnp.maximum(m1[j], t1[j] * sgn_r[j])
                r0 = []
                r1 = []
                dot0 = zero16
                dot1 = zero16
                for j in range(NCHK):
                    z0 = jnp.where(t0[j] > 0.0, t0[j],
                                   jnp.exp(t0[j]) - 1.0)
                    z1 = jnp.where(t1[j] > 0.0, t1[j],
                                   jnp.exp(t1[j]) - 1.0)
                    za0 = z0 * attn_r[j]
                    za1 = z1 * attn_r[j]
                    ra = jnp.where(za0 > 0.0, za0, jnp.exp(za0) - 1.0)
                    rb = jnp.where(za1 > 0.0, za1, jnp.exp(za1) - 1.0)
                    r0.append(ra)
                    r1.append(rb)
                    dot0 = dot0 + ra * wr_r[j]
                    dot1 = dot1 + rb * wr_r[j]
                d0 = jnp.sum(dot0) + brsc
                d1 = jnp.sum(dot1) + brsc
                s0 = 1.0 / (1.0 + jnp.exp(
                    jnp.full((LANES,), -d0, jnp.float32)))
                s1 = 1.0 / (1.0 + jnp.exp(
                    jnp.full((LANES,), -d1, jnp.float32)))
                for j in range(NCHK):
                    sl = pl.ds(LANES * j, LANES)
                    plsc.addupdate(accs.at[seg0, sl], s0 * r0[j])
                    plsc.addupdate(accs.at[seg1, sl], s1 * r1[j])
            return gcarry

        lax.fori_loop(0, CH // LANES, group_body, 0)

    load_idx(0, 0)
    load_idx(1, 1)
    wait_idx_set(0)
    fire_gather(0, 0)

    def pair_body(p, carry):
        for b in (0, 1):
            t = 2 * p + b
            nxt = 1 - b
            wait_idx_set(nxt)
            fire_gather(nxt, nxt)
            wait_gather(b, b)
            compute_chunk(b, b)
            load_idx(t + 2, b)
        return carry

    lax.fori_loop(0, nchunk // 2, pair_body, 0)
    wait_idx_set(1)
    wait_gather(0, 0)

    def merge_body(i, carry):
        for j in range(NCHK):
            sl = pl.ds(LANES * j, LANES)
            accm0[i, sl] = jnp.maximum(accm0[i, sl], accm1[i, sl])
        return carry

    lax.fori_loop(0, NSEG, merge_body, 0)
    pltpu.sync_copy(accs, outsum_hbm.at[wid])
    pltpu.sync_copy(accm0, outmax_hbm.at[wid])


def kernel(x, edge_index, edge_attr, pos, batch, W, b, attn, Wr, br):
    del edge_attr, pos
    E = edge_index.shape[1]
    N = x.shape[0]

    a = edge_index[0].astype(jnp.int32)
    b_ = edge_index[1].astype(jnp.int32)
    idx = jnp.arange(E, dtype=jnp.int32)
    first_pos = (jnp.full((N,), 2 * E, jnp.int32)
                 .at[a].min(2 * idx).at[b_].min(2 * idx + 1))
    lo = jnp.minimum(a, b_)
    hi = jnp.maximum(a, b_)
    key_s = jnp.sort(lo * N + hi)
    keep = jnp.concatenate([jnp.ones((1,), bool), key_s[1:] != key_s[:-1]])
    lo_s = key_s // N
    hi_s = key_s - lo_s * N
    batch32 = batch.astype(jnp.int32)
    node_ids = jnp.arange(N, dtype=jnp.int32)
    iso = first_pos == 2 * E
    total = E + N
    epw = -(-total // (NW * 2 * CH)) * 2 * CH
    npad = NW * epw - total + 2 * CH
    pad_idx = (jnp.arange(npad, dtype=jnp.int32) * 97) % N
    src_all = jnp.concatenate([lo_s, node_ids, pad_idx])
    dst_all = jnp.concatenate([hi_s, node_ids, pad_idx])
    valid_all = jnp.concatenate(
        [keep.astype(jnp.int32), iso.astype(jnp.int32),
         jnp.zeros((npad,), jnp.int32)])

    h = pl.pallas_call(
        _h_matmul_kernel,
        out_shape=jax.ShapeDtypeStruct((N, DIM), jnp.float32),
    )(x, W, b.reshape(1, DIM))

    nchunk = epw // CH
    mesh = plsc.VectorSubcoreMesh(
        core_axis_name="c", subcore_axis_name="s", num_cores=2,
        num_subcores=16)
    sc_fn = functools.partial(
        pl.kernel,
        out_type=[jax.ShapeDtypeStruct((NW, NSEG, DIM), jnp.float32),
                  jax.ShapeDtypeStruct((NW, NSEG, DIM), jnp.float32)],
        mesh=mesh,
        scratch_types=[
            pltpu.VMEM((CH,), jnp.int32),
            pltpu.VMEM((CH,), jnp.int32),
            pltpu.VMEM((CH,), jnp.int32),
            pltpu.VMEM((CH,), jnp.int32),
            pltpu.VMEM((CH,), jnp.int32),
            pltpu.VMEM((CH,), jnp.int32),
            pltpu.VMEM((CH, DIM), jnp.float32),
            pltpu.VMEM((CH, DIM), jnp.float32),
            pltpu.VMEM((CH, DIM), jnp.float32),
            pltpu.VMEM((CH, DIM), jnp.float32),
            pltpu.VMEM((NSEG, DIM), jnp.float32),
            pltpu.VMEM((NSEG, DIM), jnp.float32),
            pltpu.VMEM((NSEG, DIM), jnp.float32),
            pltpu.VMEM((DIM,), jnp.float32),
            pltpu.VMEM((DIM,), jnp.float32),
            pltpu.VMEM((LANES,), jnp.float32),
            pltpu.VMEM((N,), jnp.int32),
            pltpu.VMEM((N,), jnp.int32),
            pltpu.SemaphoreType.DMA,
            pltpu.SemaphoreType.DMA,
            pltpu.SemaphoreType.DMA,
            pltpu.SemaphoreType.DMA,
            pltpu.SemaphoreType.DMA,
            pltpu.SemaphoreType.DMA,
        ],
        compiler_params=pltpu.CompilerParams(needs_layout_passes=False),
    )(functools.partial(_sc_edge_kernel, nchunk))
    attn_r = attn.reshape(DIM).astype(jnp.float32)
    wr_r = Wr.reshape(DIM).astype(jnp.float32)
    br_p = jnp.broadcast_to(br.astype(jnp.float32), (LANES,))
    psum, pmax = sc_fn(h, src_all, dst_all, valid_all, attn_r, wr_r, br_p,
                       first_pos, batch32)

    out = pl.pallas_call(
        _combine_kernel,
        out_shape=jax.ShapeDtypeStruct((NSEG - 1, 2 * DIM), jnp.float32),
    )(psum, pmax, attn.reshape(1, DIM).astype(jnp.float32))
    return out


# speedup vs baseline: 1.1172x; 1.0370x over previous
import functools

import jax
import jax.numpy as jnp
from jax import lax
from jax.experimental import pallas as pl
from jax.experimental.pallas import tpu as pltpu
from jax.experimental.pallas import tpu_sc as plsc

DIM = 128
NSEG = 65
NW = 32
CH = 128
LANES = 16
NCHK = DIM // LANES


def _h_matmul_kernel(x_ref, w_ref, b_ref, o_ref):
    o_ref[...] = lax.dot_general(
        x_ref[...], w_ref[...], (((1,), (1,)), ((), ())),
        preferred_element_type=jnp.float32) + b_ref[...]


def _combine_kernel(ps_ref, pm_ref, attn_ref, o_ref):
    s = jnp.sum(ps_ref[...], axis=0)
    m = jnp.max(pm_ref[...], axis=0)
    attn = attn_ref[...]
    tstar = m * jnp.where(attn >= 0.0, 1.0, -1.0)
    z = jnp.where(tstar > 0.0, tstar, jnp.exp(tstar) - 1.0)
    za = z * attn
    r = jnp.where(za > 0.0, za, jnp.exp(za) - 1.0)
    o_ref[...] = jnp.concatenate([s[:NSEG - 1], r[:NSEG - 1]], axis=1)


def _sc_edge_kernel(nchunk, h_hbm, src_hbm, dst_hbm, seg_hbm, attn_hbm,
                    wr_hbm, br_hbm, fp_hbm, batch_hbm, outsum_hbm, outmax_hbm,
                    srcv0, dstv0, segv0, srcv1, dstv1, segv1,
                    rs0, rd0, rs1, rd1, accs, accm0, accm1,
                    attnv, wrv, brv, fpv, batchv,
                    semS0, semD0, semS1, semD1, semI0, semI1):
    cid = lax.axis_index("c")
    sid = lax.axis_index("s")
    wid = sid * 2 + cid
    epw = nchunk * CH
    base = wid * epw

    pltpu.sync_copy(attn_hbm, attnv)
    pltpu.sync_copy(wr_hbm, wrv)
    pltpu.sync_copy(br_hbm, brv)
    pltpu.sync_copy(fp_hbm, fpv)
    pltpu.sync_copy(batch_hbm, batchv)

    zero16 = jnp.zeros((LANES,), jnp.float32)
    ninf16 = jnp.full((LANES,), -jnp.inf, jnp.float32)

    def init_body(i, carry):
        for j in range(NCHK):
            sl = pl.ds(LANES * j, LANES)
            accs[i, sl] = zero16
            accm0[i, sl] = ninf16
            accm1[i, sl] = ninf16
        return carry

    lax.fori_loop(0, NSEG, init_body, 0)
    brsc = brv[...][0]
    attn_r = [attnv[pl.ds(LANES * j, LANES)] for j in range(NCHK)]
    wr_r = [wrv[pl.ds(LANES * j, LANES)] for j in range(NCHK)]
    sgn_r = [jnp.where(a >= 0.0, 1.0, -1.0).astype(jnp.float32)
             for a in attn_r]

    idx_bufs = ((srcv0, dstv0, segv0), (srcv1, dstv1, segv1))
    dat_bufs = ((rs0, rd0), (rs1, rd1))
    dat_sems = ((semS0, semD0), (semS1, semD1))
    idx_sems = (semI0, semI1)

    def load_idx(t, iset):
        off = base + t * CH
        sv, dv, gv = idx_bufs[iset]
        pltpu.async_copy(src_hbm.at[pl.ds(off, CH)], sv, idx_sems[iset])
        pltpu.async_copy(dst_hbm.at[pl.ds(off, CH)], dv, idx_sems[iset])
        pltpu.async_copy(seg_hbm.at[pl.ds(off, CH)], gv, idx_sems[iset])

    def fire_gather(dset, iset):
        sv, dv, _ = idx_bufs[iset]
        rs, rd = dat_bufs[dset]
        sA, sB = dat_sems[dset]
        pltpu.async_copy(h_hbm.at[sv], rs, sA)
        pltpu.async_copy(h_hbm.at[dv], rd, sB)

    def wait_gather(dset, iset):
        sv, dv, _ = idx_bufs[iset]
        rs, rd = dat_bufs[dset]
        sA, sB = dat_sems[dset]
        pltpu.make_async_copy(h_hbm.at[sv], rs, sA).wait()
        pltpu.make_async_copy(h_hbm.at[dv], rd, sB).wait()

    def wait_idx_set(iset):
        sv, dv, gv = idx_bufs[iset]
        off = pl.ds(0, CH)
        pltpu.make_async_copy(src_hbm.at[off], sv, idx_sems[iset]).wait()
        pltpu.make_async_copy(dst_hbm.at[off], dv, idx_sems[iset]).wait()
        pltpu.make_async_copy(seg_hbm.at[off], gv, idx_sems[iset]).wait()

    def compute_chunk(dset, iset):
        rs, rd = dat_bufs[dset]
        sv_b, dv_b, gv = idx_bufs[iset]

        def group_body(g, gcarry):
            gsl = pl.ds(g * LANES, LANES)
            lovec = sv_b[gsl]
            hivec = dv_b[gsl]
            vvec = gv[gsl]
            fpl = plsc.load_gather(fpv, [lovec])
            fph = plsc.load_gather(fpv, [hivec])
            uvec = jnp.where(fpl <= fph, lovec, hivec)
            segb = plsc.load_gather(batchv, [uvec])
            segvec = jnp.where(vvec != 0, segb, NSEG - 1)
            for p in range(LANES // 2):
                e0 = g * LANES + 2 * p
                e1 = e0 + 1
                seg0 = segvec[2 * p]
                seg1 = segvec[2 * p + 1]
                t0 = []
                t1 = []
                for j in range(NCHK):
                    sl = pl.ds(LANES * j, LANES)
                    t0.append(rs[e0, sl] + rd[e0, sl])
                    t1.append(rs[e1, sl] + rd[e1, sl])
                m0 = [accm0[seg0, pl.ds(LANES * j, LANES)]
                      for j in range(NCHK)]
                m1 = [accm1[seg1, pl.ds(LANES * j, LANES)]
                      for j in range(NCHK)]
                for j in range(NCHK):
                    sl = pl.ds(LANES * j, LANES)
                    accm0[seg0, sl] = jnp.maximum(m0[j], t0[j] * sgn_r[j])
                    accm1[seg1, sl] = jnp.maximum(m1[j], t1[j] * sgn_r[j])
                r0 = []
                r1 = []
                dot0 = zero16
                dot1 = zero16
                for j in range(NCHK):
                    z0 = jnp.where(t0[j] > 0.0, t0[j],
                                   jnp.exp(t0[j]) - 1.0)
                    z1 = jnp.where(t1[j] > 0.0, t1[j],
                                   jnp.exp(t1[j]) - 1.0)
                    za0 = z0 * attn_r[j]
                    za1 = z1 * attn_r[j]
                    ra = jnp.where(za0 > 0.0, za0, jnp.exp(za0) - 1.0)
                    rb = jnp.where(za1 > 0.0, za1, jnp.exp(za1) - 1.0)
                    r0.append(ra)
                    r1.append(rb)
                    dot0 = dot0 + ra * wr_r[j]
                    dot1 = dot1 + rb * wr_r[j]
                d0 = jnp.sum(dot0) + brsc
                d1 = jnp.sum(dot1) + brsc
                s0 = 1.0 / (1.0 + jnp.exp(
                    jnp.full((LANES,), -d0, jnp.float32)))
                s1 = 1.0 / (1.0 + jnp.exp(
                    jnp.full((LANES,), -d1, jnp.float32)))
                for j in range(NCHK):
                    sl = pl.ds(LANES * j, LANES)
                    plsc.addupdate(accs.at[seg0, sl], s0 * r0[j])
                    plsc.addupdate(accs.at[seg1, sl], s1 * r1[j])
            return gcarry

        lax.fori_loop(0, CH // LANES, group_body, 0)

    load_idx(0, 0)
    load_idx(1, 1)
    wait_idx_set(0)
    fire_gather(0, 0)

    def pair_body(p, carry):
        for b in (0, 1):
            t = 2 * p + b
            nxt = 1 - b
            wait_idx_set(nxt)
            fire_gather(nxt, nxt)
            wait_gather(b, b)
            compute_chunk(b, b)
            load_idx(t + 2, b)
        return carry

    lax.fori_loop(0, nchunk // 2, pair_body, 0)
    wait_idx_set(1)
    wait_gather(0, 0)

    def merge_body(i, carry):
        for j in range(NCHK):
            sl = pl.ds(LANES * j, LANES)
            accm0[i, sl] = jnp.maximum(accm0[i, sl], accm1[i, sl])
        return carry

    lax.fori_loop(0, NSEG, merge_body, 0)
    pltpu.sync_copy(accs, outsum_hbm.at[wid])
    pltpu.sync_copy(accm0, outmax_hbm.at[wid])


def kernel(x, edge_index, edge_attr, pos, batch, W, b, attn, Wr, br):
    del edge_attr, pos
    E = edge_index.shape[1]
    N = x.shape[0]

    a = edge_index[0].astype(jnp.int32)
    b_ = edge_index[1].astype(jnp.int32)
    idx = jnp.arange(E, dtype=jnp.int32)
    first_pos = (jnp.full((N,), 2 * E, jnp.int32)
                 .at[jnp.concatenate([a, b_])]
                 .min(jnp.concatenate([2 * idx, 2 * idx + 1])))
    lo = jnp.minimum(a, b_)
    hi = jnp.maximum(a, b_)
    key_s = jnp.sort(lo * N + hi)
    keep = jnp.concatenate([jnp.ones((1,), bool), key_s[1:] != key_s[:-1]])
    lo_s = key_s // N
    hi_s = key_s - lo_s * N
    batch32 = batch.astype(jnp.int32)
    node_ids = jnp.arange(N, dtype=jnp.int32)
    iso = first_pos == 2 * E
    total = E + N
    epw = -(-total // (NW * 2 * CH)) * 2 * CH
    npad = NW * epw - total + 2 * CH
    pad_idx = (jnp.arange(npad, dtype=jnp.int32) * 97) % N
    src_all = jnp.concatenate([lo_s, node_ids, pad_idx])
    dst_all = jnp.concatenate([hi_s, node_ids, pad_idx])
    valid_all = jnp.concatenate(
        [keep.astype(jnp.int32), iso.astype(jnp.int32),
         jnp.zeros((npad,), jnp.int32)])

    h = pl.pallas_call(
        _h_matmul_kernel,
        out_shape=jax.ShapeDtypeStruct((N, DIM), jnp.float32),
    )(x, W, b.reshape(1, DIM))

    nchunk = epw // CH
    mesh = plsc.VectorSubcoreMesh(
        core_axis_name="c", subcore_axis_name="s", num_cores=2,
        num_subcores=16)
    sc_fn = functools.partial(
        pl.kernel,
        out_type=[jax.ShapeDtypeStruct((NW, NSEG, DIM), jnp.float32),
                  jax.ShapeDtypeStruct((NW, NSEG, DIM), jnp.float32)],
        mesh=mesh,
        scratch_types=[
            pltpu.VMEM((CH,), jnp.int32),
            pltpu.VMEM((CH,), jnp.int32),
            pltpu.VMEM((CH,), jnp.int32),
            pltpu.VMEM((CH,), jnp.int32),
            pltpu.VMEM((CH,), jnp.int32),
            pltpu.VMEM((CH,), jnp.int32),
            pltpu.VMEM((CH, DIM), jnp.float32),
            pltpu.VMEM((CH, DIM), jnp.float32),
            pltpu.VMEM((CH, DIM), jnp.float32),
            pltpu.VMEM((CH, DIM), jnp.float32),
            pltpu.VMEM((NSEG, DIM), jnp.float32),
            pltpu.VMEM((NSEG, DIM), jnp.float32),
            pltpu.VMEM((NSEG, DIM), jnp.float32),
            pltpu.VMEM((DIM,), jnp.float32),
            pltpu.VMEM((DIM,), jnp.float32),
            pltpu.VMEM((LANES,), jnp.float32),
            pltpu.VMEM((N,), jnp.int32),
            pltpu.VMEM((N,), jnp.int32),
            pltpu.SemaphoreType.DMA,
            pltpu.SemaphoreType.DMA,
            pltpu.SemaphoreType.DMA,
            pltpu.SemaphoreType.DMA,
            pltpu.SemaphoreType.DMA,
            pltpu.SemaphoreType.DMA,
        ],
        compiler_params=pltpu.CompilerParams(needs_layout_passes=False),
    )(functools.partial(_sc_edge_kernel, nchunk))
    attn_r = attn.reshape(DIM).astype(jnp.float32)
    wr_r = Wr.reshape(DIM).astype(jnp.float32)
    br_p = jnp.broadcast_to(br.astype(jnp.float32), (LANES,))
    psum, pmax = sc_fn(h, src_all, dst_all, valid_all, attn_r, wr_r, br_p,
                       first_pos, batch32)

    out = pl.pallas_call(
        _combine_kernel,
        out_shape=jax.ShapeDtypeStruct((NSEG - 1, 2 * DIM), jnp.float32),
    )(psum, pmax, attn.reshape(1, DIM).astype(jnp.float32))
    return out
